# baseline (speedup 1.0000x reference)
import jax
import jax.numpy as jnp
from jax.experimental import pallas as pl

MODAL_NODES = 2
STEP_DOMAIN = 0

_BLOCK_R = 1024


def _adj_block_kernel(o_ref, *, block_r, b):
    i = pl.program_id(0)
    r0 = i * block_r
    rows = jax.lax.broadcasted_iota(jnp.int32, o_ref.shape, 0) + r0
    cols = jax.lax.broadcasted_iota(jnp.int32, o_ref.shape, 1)
    hit = (rows == cols) | ((rows >= b) & (cols >= b))
    o_ref[...] = hit.astype(jnp.float32)


def kernel(x, step):
    del step
    B, _ = x.shape
    N = B + MODAL_NODES * (STEP_DOMAIN + 1) + 1 + STEP_DOMAIN
    grid = (pl.cdiv(N, _BLOCK_R),)
    import functools
    body = functools.partial(_adj_block_kernel, block_r=_BLOCK_R, b=B)
    return pl.pallas_call(
        body,
        grid=grid,
        out_specs=pl.BlockSpec((_BLOCK_R, N), lambda i: (i, 0)),
        out_shape=jax.ShapeDtypeStruct((N, N), jnp.float32),
    )()


# speedup vs baseline: 1.1119x; 1.1119x over previous
import jax
import jax.numpy as jnp
from jax.experimental import pallas as pl

MODAL_NODES = 2
STEP_DOMAIN = 0

_BLOCK_R = 256


def _adj_block_kernel(o_ref, *, block_r, b):
    i = pl.program_id(0)
    r0 = i * block_r
    rows = jax.lax.broadcasted_iota(jnp.int32, o_ref.shape, 0) + r0
    cols = jax.lax.broadcasted_iota(jnp.int32, o_ref.shape, 1)
    hit = (rows == cols) | ((rows >= b) & (cols >= b))
    o_ref[...] = hit.astype(jnp.float32)


def kernel(x, step):
    del step
    B, _ = x.shape
    N = B + MODAL_NODES * (STEP_DOMAIN + 1) + 1 + STEP_DOMAIN
    grid = (pl.cdiv(N, _BLOCK_R),)
    import functools
    body = functools.partial(_adj_block_kernel, block_r=_BLOCK_R, b=B)
    return pl.pallas_call(
        body,
        grid=grid,
        out_specs=pl.BlockSpec((_BLOCK_R, N), lambda i: (i, 0)),
        out_shape=jax.ShapeDtypeStruct((N, N), jnp.float32),
    )()
